# Initial kernel scaffold; baseline (speedup 1.0000x reference)
import jax
import jax.numpy as jnp
from jax.experimental import pallas as pl


def kernel(mol_x, mol_edge_index, edge_index, W1m, b1m, W2m, b2m, W3m, b3m, W1, b1, W2, b2):
    raise NotImplementedError("write your pallas kernel here")


# speedup vs baseline: 25.3236x; 25.3236x over previous
import functools

import jax
import jax.numpy as jnp
from jax import lax
from jax.experimental import pallas as pl
from jax.experimental.pallas import tpu as pltpu
from jax.experimental.pallas import tpu_sc as plsc

_MB = 8
_SC_CORES = 2
_SC_TILES = 16
_DEG_W = 128


def _mol_body(kginv_ref, x_ref, ei_ref, w1_ref, b1_ref, w2_ref, b2_ref,
              w3_ref, b3_ref, zs_ref):
    mb, na, f = x_ref.shape
    me = ei_ref.shape[2]
    r = mb * na
    e = mb * me

    x = x_ref[...].reshape(r, f)
    ei = ei_ref[...]
    src = ei[:, 0, :]
    dst = ei[:, 1, :]

    loc = (lax.broadcasted_iota(jnp.int32, (mb, me, r), 2)
           - na * lax.broadcasted_iota(jnp.int32, (mb, me, r), 0))
    oh_s = (src[:, :, None] == loc).astype(jnp.bfloat16).reshape(e, r)
    oh_d = (dst[:, :, None] == loc).astype(jnp.bfloat16).reshape(e, r)
    a = lax.dot_general(oh_d, oh_s, (((0,), (0,)), ((), ())),
                        preferred_element_type=jnp.float32)
    deg = jnp.sum(a, axis=1, keepdims=True) + 1.0
    inv = lax.rsqrt(deg)
    eye = (lax.broadcasted_iota(jnp.int32, (r, r), 0)
           == lax.broadcasted_iota(jnp.int32, (r, r), 1)).astype(jnp.float32)
    ai = a + eye

    def agg(h):
        return inv * lax.dot_general(ai, inv * h, (((1,), (0,)), ((), ())),
                                     preferred_element_type=jnp.float32)

    def mm(p, w_ref):
        return lax.dot_general(p, w_ref[...], (((1,), (0,)), ((), ())),
                               preferred_element_type=jnp.float32)

    h1 = jnp.maximum(mm(agg(x), w1_ref) + b1_ref[...], 0.0)
    h2 = jnp.maximum(mm(agg(h1), w2_ref) + b2_ref[...], 0.0)
    h3 = agg(mm(h2, w3_ref)) + b3_ref[...]

    rows = jnp.concatenate(
        [jnp.min(h3[m * na:(m + 1) * na, :], axis=0, keepdims=True)
         * kginv_ref[m:m + 1, :] for m in range(mb)], axis=0)
    zs_ref[...] = rows


def _mol_call(kginv, mol_x, mol_ei, w1, b1, w2, b2, w3, b3):
    n, na, f = mol_x.shape
    me = mol_ei.shape[2]
    c2 = w1.shape[1]
    return pl.pallas_call(
        _mol_body,
        grid=(n // _MB,),
        in_specs=[
            pl.BlockSpec((_MB, 1), lambda i: (i, 0)),
            pl.BlockSpec((_MB, na, f), lambda i: (i, 0, 0)),
            pl.BlockSpec((_MB, 2, me), lambda i: (i, 0, 0)),
            pl.BlockSpec((f, c2), lambda i: (0, 0)),
            pl.BlockSpec((1, c2), lambda i: (0, 0)),
            pl.BlockSpec((c2, c2), lambda i: (0, 0)),
            pl.BlockSpec((1, c2), lambda i: (0, 0)),
            pl.BlockSpec((c2, f), lambda i: (0, 0)),
            pl.BlockSpec((1, f), lambda i: (0, 0)),
        ],
        out_specs=pl.BlockSpec((_MB, f), lambda i: (i, 0)),
        out_shape=jax.ShapeDtypeStruct((n, f), jnp.float32),
    )(kginv, mol_x, mol_ei, w1, b1.reshape(1, -1), w2, b2.reshape(1, -1),
      w3, b3.reshape(1, -1))


def _inv_body(degt_ref, out_ref):
    d = degt_ref[:, 0:1] + degt_ref[:, 1:2] + 1.0
    out_ref[...] = lax.rsqrt(d)


def _inv_call(degt):
    n = degt.shape[0]
    return pl.pallas_call(
        _inv_body,
        out_shape=jax.ShapeDtypeStruct((n, 1), jnp.float32),
    )(degt)


def _mid_body(kginv_ref, tp_ref, w1_ref, b1_ref, w2_ref, out_ref):
    inv = kginv_ref[...]
    tp = tp_ref[...]
    t = tp[0] + tp[1]
    h1 = jnp.maximum(
        lax.dot_general(inv * t, w1_ref[...], (((1,), (0,)), ((), ())),
                        preferred_element_type=jnp.float32) + b1_ref[...], 0.0)
    out_ref[...] = inv * lax.dot_general(
        h1, w2_ref[...], (((1,), (0,)), ((), ())),
        preferred_element_type=jnp.float32)


def _mid_call(kginv, tp, w1, b1, w2):
    _, n, f = tp.shape
    c2 = w1.shape[1]
    rb = 256
    return pl.pallas_call(
        _mid_body,
        grid=(n // rb,),
        in_specs=[
            pl.BlockSpec((rb, 1), lambda i: (i, 0)),
            pl.BlockSpec((2, rb, f), lambda i: (0, i, 0)),
            pl.BlockSpec((f, c2), lambda i: (0, 0)),
            pl.BlockSpec((1, c2), lambda i: (0, 0)),
            pl.BlockSpec((c2, f), lambda i: (0, 0)),
        ],
        out_specs=pl.BlockSpec((rb, f), lambda i: (i, 0)),
        out_shape=jax.ShapeDtypeStruct((n, f), jnp.float32),
    )(kginv, tp, w1, b1.reshape(1, -1), w2)


def _fin_body(kginv_ref, tp_ref, b2_ref, out_ref):
    tp = tp_ref[...]
    out_ref[...] = kginv_ref[...] * (tp[0] + tp[1]) + b2_ref[...]


def _fin_call(kginv, tp, b2):
    _, n, f = tp.shape
    rb = 256
    return pl.pallas_call(
        _fin_body,
        grid=(n // rb,),
        in_specs=[
            pl.BlockSpec((rb, 1), lambda i: (i, 0)),
            pl.BlockSpec((2, rb, f), lambda i: (0, i, 0)),
            pl.BlockSpec((1, f), lambda i: (0, 0)),
        ],
        out_specs=pl.BlockSpec((rb, f), lambda i: (i, 0)),
        out_shape=jax.ShapeDtypeStruct((n, f), jnp.float32),
    )(kginv, tp, b2.reshape(1, -1))


def _deg_call(dst, n):
    e_kg = dst.shape[0]
    nw = _SC_CORES * _SC_TILES
    ept = e_kg // nw
    ch = 128
    nch = ept // ch
    rpt = n // _SC_TILES
    mesh = plsc.VectorSubcoreMesh(core_axis_name="c", subcore_axis_name="s")

    @functools.partial(
        pl.kernel,
        out_type=jax.ShapeDtypeStruct((_SC_CORES, n, _DEG_W), jnp.float32),
        mesh=mesh,
        scratch_types=[
            pltpu.VMEM((nch, ch), jnp.int32),
            pltpu.VMEM((ch, _DEG_W), jnp.float32),
            pltpu.VMEM_SHARED((n, _DEG_W), jnp.float32),
        ],
    )
    def deg_kernel(dst_hbm, zeros_hbm, ones_hbm, out_hbm, idx_v, ones_v, acc_sh):
        c = lax.axis_index("c")
        s = lax.axis_index("s")
        wid = c * _SC_TILES + s

        pltpu.sync_copy(ones_hbm, ones_v)
        pltpu.sync_copy(dst_hbm.at[wid], idx_v)
        pltpu.sync_copy(zeros_hbm, acc_sh.at[pl.ds(s * rpt, rpt)])
        plsc.subcore_barrier()

        def body(j, _):
            pltpu.sync_copy(ones_v, acc_sh.at[idx_v.at[j]], add=True)
            return 0

        lax.fori_loop(0, nch, body, 0)
        plsc.subcore_barrier()
        pltpu.sync_copy(acc_sh.at[pl.ds(s * rpt, rpt)],
                        out_hbm.at[c, pl.ds(s * rpt, rpt)])

    zeros = jnp.zeros((rpt, _DEG_W), jnp.float32)
    ones = jnp.ones((ch, _DEG_W), jnp.float32)
    dst3 = dst.reshape(nw, nch, ch)
    return deg_kernel(dst3, zeros, ones)


def _scatter_call(table, src, dst):
    n, f = table.shape
    e_kg = src.shape[0]
    nw = _SC_CORES * _SC_TILES
    ept = e_kg // nw
    ch = 128
    nch = ept // ch
    rpt = n // _SC_TILES
    mesh = plsc.VectorSubcoreMesh(core_axis_name="c", subcore_axis_name="s")

    @functools.partial(
        pl.kernel,
        out_type=jax.ShapeDtypeStruct((_SC_CORES, n, f), jnp.float32),
        mesh=mesh,
        scratch_types=[
            pltpu.VMEM((ch,), jnp.int32),
            pltpu.VMEM((ch,), jnp.int32),
            pltpu.VMEM((ch, f), jnp.float32),
            pltpu.VMEM_SHARED((n, f), jnp.float32),
            pltpu.SemaphoreType.DMA,
        ],
    )
    def scat_kernel(tab_hbm, src_hbm, dst_hbm, zeros_hbm, out_hbm,
                    si_v, di_v, rows_v, acc_sh, sem):
        c = lax.axis_index("c")
        s = lax.axis_index("s")
        wid = c * _SC_TILES + s

        @pl.when(c == 0)
        def _():
            pltpu.sync_copy(tab_hbm.at[pl.ds(s * rpt, rpt)],
                            acc_sh.at[pl.ds(s * rpt, rpt)])

        @pl.when(c != 0)
        def _():
            pltpu.sync_copy(zeros_hbm, acc_sh.at[pl.ds(s * rpt, rpt)])

        plsc.subcore_barrier()

        def body(j, _):
            base = wid * ept + j * ch
            pltpu.sync_copy(src_hbm.at[pl.ds(base, ch)], si_v)
            pltpu.async_copy(tab_hbm.at[si_v], rows_v, sem).wait()
            pltpu.sync_copy(dst_hbm.at[pl.ds(base, ch)], di_v)
            pltpu.sync_copy(rows_v, acc_sh.at[di_v], add=True)
            return 0

        lax.fori_loop(0, nch, body, 0)
        plsc.subcore_barrier()
        pltpu.sync_copy(acc_sh.at[pl.ds(s * rpt, rpt)],
                        out_hbm.at[c, pl.ds(s * rpt, rpt)])

    zeros = jnp.zeros((rpt, f), jnp.float32)
    return scat_kernel(table, src, dst, zeros)


def kernel(mol_x, mol_edge_index, edge_index, W1m, b1m, W2m, b2m, W3m, b3m,
           W1, b1, W2, b2):
    n = mol_x.shape[0]
    src = edge_index[0]
    dst = edge_index[1]

    degw = _deg_call(dst, n)
    degt = jnp.transpose(degw[:, :, 0])
    kginv = _inv_call(degt)

    zs1 = _mol_call(kginv, mol_x, mol_edge_index,
                    W1m, b1m, W2m, b2m, W3m, b3m)
    t1p = _scatter_call(zs1, src, dst)
    zs2 = _mid_call(kginv, t1p, W1, b1, W2)
    t2p = _scatter_call(zs2, src, dst)
    return _fin_call(kginv, t2p, b2)


# speedup vs baseline: 48.1532x; 1.9015x over previous
import functools

import jax
import jax.numpy as jnp
from jax import lax
from jax.experimental import pallas as pl
from jax.experimental.pallas import tpu as pltpu
from jax.experimental.pallas import tpu_sc as plsc

_MB = 32
_SC_CORES = 2
_SC_TILES = 16
_DEG_W = 128


_SB = 8


def _mol_body(kginv_ref, x_ref, ei_ref, w1_ref, b1_ref, w2_ref, b2_ref,
              w3_ref, b3_ref, zs_ref):
    mb, na, f = x_ref.shape
    me = ei_ref.shape[2]
    mbs = mb // _SB
    r = mbs * na
    e = mbs * me

    x = x_ref[...].reshape(mb * na, f)
    ei = ei_ref[...]

    eib = ei.astype(jnp.bfloat16)
    loc = (lax.broadcasted_iota(jnp.int32, (mbs, me, r), 2)
           - na * lax.broadcasted_iota(jnp.int32, (mbs, me, r), 0)
           ).astype(jnp.bfloat16)
    eye = (lax.broadcasted_iota(jnp.int32, (r, r), 0)
           == lax.broadcasted_iota(jnp.int32, (r, r), 1)).astype(jnp.float32)
    ps, invs = [], []
    for b in range(_SB):
        sb = eib[b * mbs:(b + 1) * mbs, 0, :]
        db = eib[b * mbs:(b + 1) * mbs, 1, :]
        oh_s = (sb[:, :, None] == loc).astype(jnp.bfloat16).reshape(e, r)
        oh_d = (db[:, :, None] == loc).astype(jnp.bfloat16).reshape(e, r)
        a = lax.dot_general(oh_d, oh_s, (((0,), (0,)), ((), ())),
                            preferred_element_type=jnp.float32)
        deg = jnp.sum(a, axis=1, keepdims=True) + 1.0
        inv = lax.rsqrt(deg)
        ps.append(((a + eye) * inv).astype(jnp.bfloat16))
        invs.append(inv)

    def agg(h):
        outs = []
        for b in range(_SB):
            hs = (invs[b] * h[b * r:(b + 1) * r, :]).astype(jnp.bfloat16)
            outs.append(lax.dot_general(ps[b], hs, (((1,), (0,)), ((), ())),
                                        preferred_element_type=jnp.float32))
        return jnp.concatenate(outs, axis=0)

    def mm(p, w_ref):
        return lax.dot_general(p.astype(jnp.bfloat16), w_ref[...],
                               (((1,), (0,)), ((), ())),
                               preferred_element_type=jnp.float32)

    h1 = jnp.maximum(mm(agg(x), w1_ref) + b1_ref[...], 0.0)
    h2 = jnp.maximum(mm(agg(h1), w2_ref) + b2_ref[...], 0.0)
    h3 = agg(mm(h2, w3_ref)) + b3_ref[...]

    rows = jnp.concatenate(
        [jnp.min(h3[m * na:(m + 1) * na, :], axis=0, keepdims=True)
         * kginv_ref[m:m + 1, :] for m in range(mb)], axis=0)
    zs_ref[...] = rows


def _mol_call(kginv, mol_x, mol_ei, w1, b1, w2, b2, w3, b3):
    n, na, f = mol_x.shape
    me = mol_ei.shape[2]
    c2 = w1.shape[1]
    return pl.pallas_call(
        _mol_body,
        grid=(n // _MB,),
        in_specs=[
            pl.BlockSpec((_MB, 1), lambda i: (i, 0)),
            pl.BlockSpec((_MB, na, f), lambda i: (i, 0, 0)),
            pl.BlockSpec((_MB, 2, me), lambda i: (i, 0, 0)),
            pl.BlockSpec((f, c2), lambda i: (0, 0)),
            pl.BlockSpec((1, c2), lambda i: (0, 0)),
            pl.BlockSpec((c2, c2), lambda i: (0, 0)),
            pl.BlockSpec((1, c2), lambda i: (0, 0)),
            pl.BlockSpec((c2, f), lambda i: (0, 0)),
            pl.BlockSpec((1, f), lambda i: (0, 0)),
        ],
        out_specs=pl.BlockSpec((_MB, f), lambda i: (i, 0)),
        out_shape=jax.ShapeDtypeStruct((n, f), jnp.float32),
    )(kginv, mol_x, mol_ei,
      w1.astype(jnp.bfloat16), b1.reshape(1, -1),
      w2.astype(jnp.bfloat16), b2.reshape(1, -1),
      w3.astype(jnp.bfloat16), b3.reshape(1, -1))


def _inv_body(degt_ref, out_ref):
    d = degt_ref[:, 0:1] + degt_ref[:, 1:2] + 1.0
    out_ref[...] = lax.rsqrt(d)


def _inv_call(degt):
    n = degt.shape[0]
    return pl.pallas_call(
        _inv_body,
        out_shape=jax.ShapeDtypeStruct((n, 1), jnp.float32),
    )(degt)


def _mid_body(kginv_ref, tp_ref, w1_ref, b1_ref, w2_ref, out_ref):
    inv = kginv_ref[...]
    tp = tp_ref[...]
    t = tp[0] + tp[1]
    h1 = jnp.maximum(
        lax.dot_general(inv * t, w1_ref[...], (((1,), (0,)), ((), ())),
                        preferred_element_type=jnp.float32) + b1_ref[...], 0.0)
    out_ref[...] = inv * lax.dot_general(
        h1, w2_ref[...], (((1,), (0,)), ((), ())),
        preferred_element_type=jnp.float32)


def _mid_call(kginv, tp, w1, b1, w2):
    _, n, f = tp.shape
    c2 = w1.shape[1]
    rb = 256
    return pl.pallas_call(
        _mid_body,
        grid=(n // rb,),
        in_specs=[
            pl.BlockSpec((rb, 1), lambda i: (i, 0)),
            pl.BlockSpec((2, rb, f), lambda i: (0, i, 0)),
            pl.BlockSpec((f, c2), lambda i: (0, 0)),
            pl.BlockSpec((1, c2), lambda i: (0, 0)),
            pl.BlockSpec((c2, f), lambda i: (0, 0)),
        ],
        out_specs=pl.BlockSpec((rb, f), lambda i: (i, 0)),
        out_shape=jax.ShapeDtypeStruct((n, f), jnp.float32),
    )(kginv, tp, w1, b1.reshape(1, -1), w2)


def _fin_body(kginv_ref, tp_ref, b2_ref, out_ref):
    tp = tp_ref[...]
    out_ref[...] = kginv_ref[...] * (tp[0] + tp[1]) + b2_ref[...]


def _fin_call(kginv, tp, b2):
    _, n, f = tp.shape
    rb = 256
    return pl.pallas_call(
        _fin_body,
        grid=(n // rb,),
        in_specs=[
            pl.BlockSpec((rb, 1), lambda i: (i, 0)),
            pl.BlockSpec((2, rb, f), lambda i: (0, i, 0)),
            pl.BlockSpec((1, f), lambda i: (0, 0)),
        ],
        out_specs=pl.BlockSpec((rb, f), lambda i: (i, 0)),
        out_shape=jax.ShapeDtypeStruct((n, f), jnp.float32),
    )(kginv, tp, b2.reshape(1, -1))


def _deg_call(dst, n):
    e_kg = dst.shape[0]
    nw = _SC_CORES * _SC_TILES
    ept = e_kg // nw
    ch = 128
    nch = ept // ch
    rpt = n // _SC_TILES
    mesh = plsc.VectorSubcoreMesh(core_axis_name="c", subcore_axis_name="s")

    @functools.partial(
        pl.kernel,
        out_type=jax.ShapeDtypeStruct((_SC_CORES, n, _DEG_W), jnp.float32),
        mesh=mesh,
        scratch_types=[
            pltpu.VMEM((nch, ch), jnp.int32),
            pltpu.VMEM((ch, _DEG_W), jnp.float32),
            pltpu.VMEM_SHARED((n, _DEG_W), jnp.float32),
        ],
    )
    def deg_kernel(dst_hbm, zeros_hbm, ones_hbm, out_hbm, idx_v, ones_v, acc_sh):
        c = lax.axis_index("c")
        s = lax.axis_index("s")
        wid = c * _SC_TILES + s

        pltpu.sync_copy(ones_hbm, ones_v)
        pltpu.sync_copy(dst_hbm.at[wid], idx_v)
        pltpu.sync_copy(zeros_hbm, acc_sh.at[pl.ds(s * rpt, rpt)])
        plsc.subcore_barrier()

        def body(j, _):
            pltpu.sync_copy(ones_v, acc_sh.at[idx_v.at[j]], add=True)
            return 0

        lax.fori_loop(0, nch, body, 0)
        plsc.subcore_barrier()
        pltpu.sync_copy(acc_sh.at[pl.ds(s * rpt, rpt)],
                        out_hbm.at[c, pl.ds(s * rpt, rpt)])

    zeros = jnp.zeros((rpt, _DEG_W), jnp.float32)
    ones = jnp.ones((ch, _DEG_W), jnp.float32)
    dst3 = dst.reshape(nw, nch, ch)
    return deg_kernel(dst3, zeros, ones)


def _scatter_call(table, src, dst):
    n, f = table.shape
    e_kg = src.shape[0]
    nw = _SC_CORES * _SC_TILES
    ept = e_kg // nw
    ch = 128
    nch = ept // ch
    rpt = n // _SC_TILES
    mesh = plsc.VectorSubcoreMesh(core_axis_name="c", subcore_axis_name="s")

    @functools.partial(
        pl.kernel,
        out_type=jax.ShapeDtypeStruct((_SC_CORES, n, f), jnp.float32),
        mesh=mesh,
        scratch_types=[
            pltpu.VMEM((ch,), jnp.int32),
            pltpu.VMEM((ch,), jnp.int32),
            pltpu.VMEM((ch, f), jnp.float32),
            pltpu.VMEM_SHARED((n, f), jnp.float32),
            pltpu.SemaphoreType.DMA,
        ],
    )
    def scat_kernel(tab_hbm, src_hbm, dst_hbm, zeros_hbm, out_hbm,
                    si_v, di_v, rows_v, acc_sh, sem):
        c = lax.axis_index("c")
        s = lax.axis_index("s")
        wid = c * _SC_TILES + s

        @pl.when(c == 0)
        def _():
            pltpu.sync_copy(tab_hbm.at[pl.ds(s * rpt, rpt)],
                            acc_sh.at[pl.ds(s * rpt, rpt)])

        @pl.when(c != 0)
        def _():
            pltpu.sync_copy(zeros_hbm, acc_sh.at[pl.ds(s * rpt, rpt)])

        plsc.subcore_barrier()

        def body(j, _):
            base = wid * ept + j * ch
            pltpu.sync_copy(src_hbm.at[pl.ds(base, ch)], si_v)
            pltpu.async_copy(tab_hbm.at[si_v], rows_v, sem).wait()
            pltpu.sync_copy(dst_hbm.at[pl.ds(base, ch)], di_v)
            pltpu.sync_copy(rows_v, acc_sh.at[di_v], add=True)
            return 0

        lax.fori_loop(0, nch, body, 0)
        plsc.subcore_barrier()
        pltpu.sync_copy(acc_sh.at[pl.ds(s * rpt, rpt)],
                        out_hbm.at[c, pl.ds(s * rpt, rpt)])

    zeros = jnp.zeros((rpt, f), jnp.float32)
    return scat_kernel(table, src, dst, zeros)


def kernel(mol_x, mol_edge_index, edge_index, W1m, b1m, W2m, b2m, W3m, b3m,
           W1, b1, W2, b2):
    n = mol_x.shape[0]
    src = edge_index[0]
    dst = edge_index[1]

    degw = _deg_call(dst, n)
    degt = jnp.transpose(degw[:, :, 0])
    kginv = _inv_call(degt)

    zs1 = _mol_call(kginv, mol_x, mol_edge_index,
                    W1m, b1m, W2m, b2m, W3m, b3m)
    t1p = _scatter_call(zs1, src, dst)
    zs2 = _mid_call(kginv, t1p, W1, b1, W2)
    t2p = _scatter_call(zs2, src, dst)
    return _fin_call(kginv, t2p, b2)


# speedup vs baseline: 57.7203x; 1.1987x over previous
import functools

import jax
import jax.numpy as jnp
from jax import lax
from jax.experimental import pallas as pl
from jax.experimental.pallas import tpu as pltpu
from jax.experimental.pallas import tpu_sc as plsc

_MB = 32
_SC_CORES = 2
_SC_TILES = 16
_DEG_W = 128


_SB = 8


def _mol_body(kginv_ref, x_ref, ei_ref, w1_ref, b1_ref, w2_ref, b2_ref,
              w3_ref, b3_ref, zs_ref):
    mb, na, f = x_ref.shape
    me = ei_ref.shape[2]
    mbs = mb // _SB
    r = mbs * na
    e = mbs * me

    x = x_ref[...].reshape(mb * na, f)
    ei = ei_ref[...]

    eib = ei.astype(jnp.bfloat16)
    loc = (lax.broadcasted_iota(jnp.int32, (mbs, me, r), 2)
           - na * lax.broadcasted_iota(jnp.int32, (mbs, me, r), 0)
           ).astype(jnp.bfloat16)
    eye = (lax.broadcasted_iota(jnp.int32, (r, r), 0)
           == lax.broadcasted_iota(jnp.int32, (r, r), 1)).astype(jnp.float32)
    ps, invs = [], []
    for b in range(_SB):
        sb = eib[b * mbs:(b + 1) * mbs, 0, :]
        db = eib[b * mbs:(b + 1) * mbs, 1, :]
        oh_s = (sb[:, :, None] == loc).astype(jnp.bfloat16).reshape(e, r)
        oh_d = (db[:, :, None] == loc).astype(jnp.bfloat16).reshape(e, r)
        a = lax.dot_general(oh_d, oh_s, (((0,), (0,)), ((), ())),
                            preferred_element_type=jnp.float32)
        deg = jnp.sum(a, axis=1, keepdims=True) + 1.0
        inv = lax.rsqrt(deg)
        ps.append(((a + eye) * inv).astype(jnp.bfloat16))
        invs.append(inv)

    def agg(h):
        outs = []
        for b in range(_SB):
            hs = (invs[b] * h[b * r:(b + 1) * r, :]).astype(jnp.bfloat16)
            outs.append(lax.dot_general(ps[b], hs, (((1,), (0,)), ((), ())),
                                        preferred_element_type=jnp.float32))
        return jnp.concatenate(outs, axis=0)

    def mm(p, w_ref):
        return lax.dot_general(p.astype(jnp.bfloat16), w_ref[...],
                               (((1,), (0,)), ((), ())),
                               preferred_element_type=jnp.float32)

    h1 = jnp.maximum(mm(agg(x), w1_ref) + b1_ref[...], 0.0)
    h2 = jnp.maximum(mm(agg(h1), w2_ref) + b2_ref[...], 0.0)
    h3 = agg(mm(h2, w3_ref)) + b3_ref[...]

    rows = jnp.concatenate(
        [jnp.min(h3[m * na:(m + 1) * na, :], axis=0, keepdims=True)
         * kginv_ref[m:m + 1, :] for m in range(mb)], axis=0)
    zs_ref[...] = rows


def _mol_call(kginv, mol_x, mol_ei, w1, b1, w2, b2, w3, b3):
    n, na, f = mol_x.shape
    me = mol_ei.shape[2]
    c2 = w1.shape[1]
    return pl.pallas_call(
        _mol_body,
        grid=(n // _MB,),
        in_specs=[
            pl.BlockSpec((_MB, 1), lambda i: (i, 0)),
            pl.BlockSpec((_MB, na, f), lambda i: (i, 0, 0)),
            pl.BlockSpec((_MB, 2, me), lambda i: (i, 0, 0)),
            pl.BlockSpec((f, c2), lambda i: (0, 0)),
            pl.BlockSpec((1, c2), lambda i: (0, 0)),
            pl.BlockSpec((c2, c2), lambda i: (0, 0)),
            pl.BlockSpec((1, c2), lambda i: (0, 0)),
            pl.BlockSpec((c2, f), lambda i: (0, 0)),
            pl.BlockSpec((1, f), lambda i: (0, 0)),
        ],
        out_specs=pl.BlockSpec((_MB, f), lambda i: (i, 0)),
        out_shape=jax.ShapeDtypeStruct((n, f), jnp.float32),
    )(kginv, mol_x, mol_ei,
      w1.astype(jnp.bfloat16), b1.reshape(1, -1),
      w2.astype(jnp.bfloat16), b2.reshape(1, -1),
      w3.astype(jnp.bfloat16), b3.reshape(1, -1))


def _inv_body(degt_ref, out_ref):
    d = degt_ref[:, 0:1] + degt_ref[:, 1:2] + 1.0
    out_ref[...] = lax.rsqrt(d)


def _inv_call(degt):
    n = degt.shape[0]
    return pl.pallas_call(
        _inv_body,
        out_shape=jax.ShapeDtypeStruct((n, 1), jnp.float32),
    )(degt)


def _mid_body(kginv_ref, tp_ref, w1_ref, b1_ref, w2_ref, out_ref):
    inv = kginv_ref[...]
    tp = tp_ref[...]
    t = tp[0] + tp[1]
    h1 = jnp.maximum(
        lax.dot_general(inv * t, w1_ref[...], (((1,), (0,)), ((), ())),
                        preferred_element_type=jnp.float32) + b1_ref[...], 0.0)
    out_ref[...] = inv * lax.dot_general(
        h1, w2_ref[...], (((1,), (0,)), ((), ())),
        preferred_element_type=jnp.float32)


def _mid_call(kginv, tp, w1, b1, w2):
    _, n, f = tp.shape
    c2 = w1.shape[1]
    rb = 256
    return pl.pallas_call(
        _mid_body,
        grid=(n // rb,),
        in_specs=[
            pl.BlockSpec((rb, 1), lambda i: (i, 0)),
            pl.BlockSpec((2, rb, f), lambda i: (0, i, 0)),
            pl.BlockSpec((f, c2), lambda i: (0, 0)),
            pl.BlockSpec((1, c2), lambda i: (0, 0)),
            pl.BlockSpec((c2, f), lambda i: (0, 0)),
        ],
        out_specs=pl.BlockSpec((rb, f), lambda i: (i, 0)),
        out_shape=jax.ShapeDtypeStruct((n, f), jnp.float32),
    )(kginv, tp, w1, b1.reshape(1, -1), w2)


def _fin_body(kginv_ref, tp_ref, b2_ref, out_ref):
    tp = tp_ref[...]
    out_ref[...] = kginv_ref[...] * (tp[0] + tp[1]) + b2_ref[...]


def _fin_call(kginv, tp, b2):
    _, n, f = tp.shape
    rb = 256
    return pl.pallas_call(
        _fin_body,
        grid=(n // rb,),
        in_specs=[
            pl.BlockSpec((rb, 1), lambda i: (i, 0)),
            pl.BlockSpec((2, rb, f), lambda i: (0, i, 0)),
            pl.BlockSpec((1, f), lambda i: (0, 0)),
        ],
        out_specs=pl.BlockSpec((rb, f), lambda i: (i, 0)),
        out_shape=jax.ShapeDtypeStruct((n, f), jnp.float32),
    )(kginv, tp, b2.reshape(1, -1))


def _deg_call(dst, n):
    e_kg = dst.shape[0]
    nw = _SC_CORES * _SC_TILES
    ept = e_kg // nw
    ch = 128
    nch = ept // ch
    rpt = n // _SC_TILES
    mesh = plsc.VectorSubcoreMesh(core_axis_name="c", subcore_axis_name="s")

    @functools.partial(
        pl.kernel,
        out_type=jax.ShapeDtypeStruct((_SC_CORES, n, _DEG_W), jnp.float32),
        mesh=mesh,
        scratch_types=[
            pltpu.VMEM((nch, ch), jnp.int32),
            pltpu.VMEM((ch, _DEG_W), jnp.float32),
            pltpu.VMEM_SHARED((n, _DEG_W), jnp.float32),
            pltpu.SemaphoreType.DMA,
        ],
    )
    def deg_kernel(dst_hbm, zeros_hbm, ones_hbm, out_hbm, idx_v, ones_v,
                   acc_sh, sem):
        c = lax.axis_index("c")
        s = lax.axis_index("s")
        wid = c * _SC_TILES + s

        pltpu.sync_copy(ones_hbm, ones_v)
        pltpu.sync_copy(dst_hbm.at[wid], idx_v)
        pltpu.sync_copy(zeros_hbm, acc_sh.at[pl.ds(s * rpt, rpt)])
        plsc.subcore_barrier()

        def body(j, _):
            pltpu.async_copy(ones_v, acc_sh.at[idx_v.at[j]], sem, add=True)
            return 0

        lax.fori_loop(0, nch, body, 0)

        def drain(j, _):
            pltpu.make_async_copy(ones_hbm, ones_v, sem).wait()
            return 0

        lax.fori_loop(0, nch, drain, 0)
        plsc.subcore_barrier()
        pltpu.sync_copy(acc_sh.at[pl.ds(s * rpt, rpt)],
                        out_hbm.at[c, pl.ds(s * rpt, rpt)])

    zeros = jnp.zeros((rpt, _DEG_W), jnp.float32)
    ones = jnp.ones((ch, _DEG_W), jnp.float32)
    dst3 = dst.reshape(nw, nch, ch)
    return deg_kernel(dst3, zeros, ones)


def _scatter_call(table, src, dst):
    n, f = table.shape
    e_kg = src.shape[0]
    nw = _SC_CORES * _SC_TILES
    ept = e_kg // nw
    ch = 128
    nch = ept // ch
    rpt = n // _SC_TILES
    mesh = plsc.VectorSubcoreMesh(core_axis_name="c", subcore_axis_name="s")

    @functools.partial(
        pl.kernel,
        out_type=jax.ShapeDtypeStruct((_SC_CORES, n, f), jnp.float32),
        mesh=mesh,
        scratch_types=[
            pltpu.VMEM((nch, ch), jnp.int32),
            pltpu.VMEM((nch, ch), jnp.int32),
            pltpu.VMEM((ch, f), jnp.float32),
            pltpu.VMEM((ch, f), jnp.float32),
            pltpu.VMEM_SHARED((n, f), jnp.float32),
            pltpu.SemaphoreType.DMA,
            pltpu.SemaphoreType.DMA,
        ],
    )
    def scat_kernel(tab_hbm, src_hbm, dst_hbm, zeros_hbm, out_hbm,
                    si_v, di_v, rows0, rows1, acc_sh, sem0, sem1):
        c = lax.axis_index("c")
        s = lax.axis_index("s")
        wid = c * _SC_TILES + s

        pltpu.sync_copy(src_hbm.at[wid], si_v)
        pltpu.sync_copy(dst_hbm.at[wid], di_v)

        @pl.when(c == 0)
        def _():
            pltpu.sync_copy(tab_hbm.at[pl.ds(s * rpt, rpt)],
                            acc_sh.at[pl.ds(s * rpt, rpt)])

        @pl.when(c != 0)
        def _():
            pltpu.sync_copy(zeros_hbm, acc_sh.at[pl.ds(s * rpt, rpt)])

        plsc.subcore_barrier()

        def gather(j, buf, sem):
            pltpu.async_copy(tab_hbm.at[si_v.at[j]], buf, sem)

        def gwait(buf, sem):
            pltpu.make_async_copy(tab_hbm.at[si_v.at[0]], buf, sem).wait()

        gather(0, rows0, sem0)

        def body(jj, _):
            j0 = 2 * jj
            gather(j0 + 1, rows1, sem1)
            gwait(rows0, sem0)
            pltpu.sync_copy(rows0, acc_sh.at[di_v.at[j0]], add=True)

            @pl.when(j0 + 2 < nch)
            def _():
                gather(j0 + 2, rows0, sem0)

            gwait(rows1, sem1)
            pltpu.sync_copy(rows1, acc_sh.at[di_v.at[j0 + 1]], add=True)
            return 0

        lax.fori_loop(0, nch // 2, body, 0)
        plsc.subcore_barrier()
        pltpu.sync_copy(acc_sh.at[pl.ds(s * rpt, rpt)],
                        out_hbm.at[c, pl.ds(s * rpt, rpt)])

    zeros = jnp.zeros((rpt, f), jnp.float32)
    src3 = src.reshape(nw, nch, ch)
    dst3 = dst.reshape(nw, nch, ch)
    return scat_kernel(table, src3, dst3, zeros)


def kernel(mol_x, mol_edge_index, edge_index, W1m, b1m, W2m, b2m, W3m, b3m,
           W1, b1, W2, b2):
    n = mol_x.shape[0]
    src = edge_index[0]
    dst = edge_index[1]

    degw = _deg_call(dst, n)
    degt = jnp.transpose(degw[:, :, 0])
    kginv = _inv_call(degt)

    zs1 = _mol_call(kginv, mol_x, mol_edge_index,
                    W1m, b1m, W2m, b2m, W3m, b3m)
    t1p = _scatter_call(zs1, src, dst)
    zs2 = _mid_call(kginv, t1p, W1, b1, W2)
    t2p = _scatter_call(zs2, src, dst)
    return _fin_call(kginv, t2p, b2)


# speedup vs baseline: 61.5091x; 1.0656x over previous
import functools

import jax
import jax.numpy as jnp
from jax import lax
from jax.experimental import pallas as pl
from jax.experimental.pallas import tpu as pltpu
from jax.experimental.pallas import tpu_sc as plsc

_MB = 32
_SC_CORES = 2
_SC_TILES = 16
_DEG_W = 128


_SB = 8


def _mol_body(kginv_ref, x_ref, ei_ref, w1_ref, b1_ref, w2_ref, b2_ref,
              w3_ref, b3_ref, zs_ref):
    mb, na, f = x_ref.shape
    me = ei_ref.shape[2]
    mbs = mb // _SB
    r = mbs * na
    e = mbs * me

    x = x_ref[...].reshape(mb * na, f)
    ei = ei_ref[...]

    eib = ei.astype(jnp.bfloat16)
    loc = (lax.broadcasted_iota(jnp.int32, (mbs, me, r), 2)
           - na * lax.broadcasted_iota(jnp.int32, (mbs, me, r), 0)
           ).astype(jnp.bfloat16)
    eye = (lax.broadcasted_iota(jnp.int32, (r, r), 0)
           == lax.broadcasted_iota(jnp.int32, (r, r), 1)).astype(jnp.float32)
    ps, invs = [], []
    for b in range(_SB):
        sb = eib[b * mbs:(b + 1) * mbs, 0, :]
        db = eib[b * mbs:(b + 1) * mbs, 1, :]
        oh_s = (sb[:, :, None] == loc).astype(jnp.int8).reshape(e, r)
        oh_d = (db[:, :, None] == loc).astype(jnp.int8).reshape(e, r)
        a = lax.dot_general(oh_d, oh_s, (((0,), (0,)), ((), ())),
                            preferred_element_type=jnp.int32
                            ).astype(jnp.float32)
        deg = jnp.sum(a, axis=1, keepdims=True) + 1.0
        inv = lax.rsqrt(deg)
        ps.append(((a + eye) * inv).astype(jnp.bfloat16))
        invs.append(inv)

    def agg(h):
        outs = []
        for b in range(_SB):
            hs = (invs[b] * h[b * r:(b + 1) * r, :]).astype(jnp.bfloat16)
            outs.append(lax.dot_general(ps[b], hs, (((1,), (0,)), ((), ())),
                                        preferred_element_type=jnp.float32))
        return jnp.concatenate(outs, axis=0)

    def mm(p, w_ref):
        return lax.dot_general(p.astype(jnp.bfloat16), w_ref[...],
                               (((1,), (0,)), ((), ())),
                               preferred_element_type=jnp.float32)

    h1 = jnp.maximum(mm(agg(x), w1_ref) + b1_ref[...], 0.0)
    h2 = jnp.maximum(mm(agg(h1), w2_ref) + b2_ref[...], 0.0)
    h3 = agg(mm(h2, w3_ref)) + b3_ref[...]

    rows = jnp.concatenate(
        [jnp.min(h3[m * na:(m + 1) * na, :], axis=0, keepdims=True)
         * kginv_ref[m:m + 1, :] for m in range(mb)], axis=0)
    zs_ref[...] = rows


def _mol_call(kginv, mol_x, mol_ei, w1, b1, w2, b2, w3, b3):
    n, na, f = mol_x.shape
    me = mol_ei.shape[2]
    c2 = w1.shape[1]
    return pl.pallas_call(
        _mol_body,
        grid=(n // _MB,),
        in_specs=[
            pl.BlockSpec((_MB, 1), lambda i: (i, 0)),
            pl.BlockSpec((_MB, na, f), lambda i: (i, 0, 0)),
            pl.BlockSpec((_MB, 2, me), lambda i: (i, 0, 0)),
            pl.BlockSpec((f, c2), lambda i: (0, 0)),
            pl.BlockSpec((1, c2), lambda i: (0, 0)),
            pl.BlockSpec((c2, c2), lambda i: (0, 0)),
            pl.BlockSpec((1, c2), lambda i: (0, 0)),
            pl.BlockSpec((c2, f), lambda i: (0, 0)),
            pl.BlockSpec((1, f), lambda i: (0, 0)),
        ],
        out_specs=pl.BlockSpec((_MB, f), lambda i: (i, 0)),
        out_shape=jax.ShapeDtypeStruct((n, f), jnp.float32),
    )(kginv, mol_x, mol_ei,
      w1.astype(jnp.bfloat16), b1.reshape(1, -1),
      w2.astype(jnp.bfloat16), b2.reshape(1, -1),
      w3.astype(jnp.bfloat16), b3.reshape(1, -1))


def _inv_body(degt_ref, out_ref):
    d = degt_ref[:, 0:1] + degt_ref[:, 1:2] + 1.0
    out_ref[...] = lax.rsqrt(d)


def _inv_call(degt):
    n = degt.shape[0]
    return pl.pallas_call(
        _inv_body,
        out_shape=jax.ShapeDtypeStruct((n, 1), jnp.float32),
    )(degt)


def _mid_body(kginv_ref, tp_ref, w1_ref, b1_ref, w2_ref, out_ref):
    inv = kginv_ref[...]
    tp = tp_ref[...]
    t = tp[0] + tp[1]
    h1 = jnp.maximum(
        lax.dot_general(inv * t, w1_ref[...], (((1,), (0,)), ((), ())),
                        preferred_element_type=jnp.float32) + b1_ref[...], 0.0)
    out_ref[...] = inv * lax.dot_general(
        h1, w2_ref[...], (((1,), (0,)), ((), ())),
        preferred_element_type=jnp.float32)


def _mid_call(kginv, tp, w1, b1, w2):
    _, n, f = tp.shape
    c2 = w1.shape[1]
    rb = 256
    return pl.pallas_call(
        _mid_body,
        grid=(n // rb,),
        in_specs=[
            pl.BlockSpec((rb, 1), lambda i: (i, 0)),
            pl.BlockSpec((2, rb, f), lambda i: (0, i, 0)),
            pl.BlockSpec((f, c2), lambda i: (0, 0)),
            pl.BlockSpec((1, c2), lambda i: (0, 0)),
            pl.BlockSpec((c2, f), lambda i: (0, 0)),
        ],
        out_specs=pl.BlockSpec((rb, f), lambda i: (i, 0)),
        out_shape=jax.ShapeDtypeStruct((n, f), jnp.float32),
    )(kginv, tp, w1, b1.reshape(1, -1), w2)


def _fin_body(kginv_ref, tp_ref, b2_ref, out_ref):
    tp = tp_ref[...]
    out_ref[...] = kginv_ref[...] * (tp[0] + tp[1]) + b2_ref[...]


def _fin_call(kginv, tp, b2):
    _, n, f = tp.shape
    rb = 256
    return pl.pallas_call(
        _fin_body,
        grid=(n // rb,),
        in_specs=[
            pl.BlockSpec((rb, 1), lambda i: (i, 0)),
            pl.BlockSpec((2, rb, f), lambda i: (0, i, 0)),
            pl.BlockSpec((1, f), lambda i: (0, 0)),
        ],
        out_specs=pl.BlockSpec((rb, f), lambda i: (i, 0)),
        out_shape=jax.ShapeDtypeStruct((n, f), jnp.float32),
    )(kginv, tp, b2.reshape(1, -1))


def _deg_call(dst, n):
    e_kg = dst.shape[0]
    nw = _SC_CORES * _SC_TILES
    ept = e_kg // nw
    ch = 128
    nch = ept // ch
    rpt = n // _SC_TILES
    mesh = plsc.VectorSubcoreMesh(core_axis_name="c", subcore_axis_name="s")

    @functools.partial(
        pl.kernel,
        out_type=jax.ShapeDtypeStruct((_SC_CORES, n, _DEG_W), jnp.float32),
        mesh=mesh,
        scratch_types=[
            pltpu.VMEM((nch, ch), jnp.int32),
            pltpu.VMEM((ch, _DEG_W), jnp.float32),
            pltpu.VMEM_SHARED((n, _DEG_W), jnp.float32),
            pltpu.SemaphoreType.DMA,
        ],
    )
    def deg_kernel(dst_hbm, zeros_hbm, ones_hbm, out_hbm, idx_v, ones_v,
                   acc_sh, sem):
        c = lax.axis_index("c")
        s = lax.axis_index("s")
        wid = c * _SC_TILES + s

        pltpu.sync_copy(ones_hbm, ones_v)
        pltpu.sync_copy(dst_hbm.at[wid], idx_v)
        pltpu.sync_copy(zeros_hbm, acc_sh.at[pl.ds(s * rpt, rpt)])
        plsc.subcore_barrier()

        def body(j, _):
            pltpu.async_copy(ones_v, acc_sh.at[idx_v.at[j]], sem, add=True)
            return 0

        lax.fori_loop(0, nch, body, 0)

        def drain(j, _):
            pltpu.make_async_copy(ones_hbm, ones_v, sem).wait()
            return 0

        lax.fori_loop(0, nch, drain, 0)
        plsc.subcore_barrier()
        pltpu.sync_copy(acc_sh.at[pl.ds(s * rpt, rpt)],
                        out_hbm.at[c, pl.ds(s * rpt, rpt)])

    zeros = jnp.zeros((rpt, _DEG_W), jnp.float32)
    ones = jnp.ones((ch, _DEG_W), jnp.float32)
    dst3 = dst.reshape(nw, nch, ch)
    return deg_kernel(dst3, zeros, ones)


def _scatter_call(table, src, dst):
    n, f = table.shape
    e_kg = src.shape[0]
    nw = _SC_CORES * _SC_TILES
    ept = e_kg // nw
    ch = 128
    nch = ept // ch
    rpt = n // _SC_TILES
    mesh = plsc.VectorSubcoreMesh(core_axis_name="c", subcore_axis_name="s")

    @functools.partial(
        pl.kernel,
        out_type=jax.ShapeDtypeStruct((_SC_CORES, n, f), jnp.float32),
        mesh=mesh,
        scratch_types=[
            pltpu.VMEM((nch, ch), jnp.int32),
            pltpu.VMEM((nch, ch), jnp.int32),
            pltpu.VMEM((ch, f), jnp.float32),
            pltpu.VMEM((ch, f), jnp.float32),
            pltpu.VMEM_SHARED((n, f), jnp.float32),
            pltpu.SemaphoreType.DMA,
            pltpu.SemaphoreType.DMA,
        ],
    )
    def scat_kernel(tab_hbm, src_hbm, dst_hbm, zeros_hbm, out_hbm,
                    si_v, di_v, rows0, rows1, acc_sh, sem0, sem1):
        c = lax.axis_index("c")
        s = lax.axis_index("s")
        wid = c * _SC_TILES + s

        pltpu.sync_copy(src_hbm.at[wid], si_v)
        pltpu.sync_copy(dst_hbm.at[wid], di_v)

        @pl.when(c == 0)
        def _():
            pltpu.sync_copy(tab_hbm.at[pl.ds(s * rpt, rpt)],
                            acc_sh.at[pl.ds(s * rpt, rpt)])

        @pl.when(c != 0)
        def _():
            pltpu.sync_copy(zeros_hbm, acc_sh.at[pl.ds(s * rpt, rpt)])

        plsc.subcore_barrier()

        def gather(j, buf, sem):
            pltpu.async_copy(tab_hbm.at[si_v.at[j]], buf, sem)

        def gwait(buf, sem):
            pltpu.make_async_copy(tab_hbm.at[si_v.at[0]], buf, sem).wait()

        gather(0, rows0, sem0)

        def body(jj, _):
            j0 = 2 * jj
            gather(j0 + 1, rows1, sem1)
            gwait(rows0, sem0)
            pltpu.sync_copy(rows0, acc_sh.at[di_v.at[j0]], add=True)

            @pl.when(j0 + 2 < nch)
            def _():
                gather(j0 + 2, rows0, sem0)

            gwait(rows1, sem1)
            pltpu.sync_copy(rows1, acc_sh.at[di_v.at[j0 + 1]], add=True)
            return 0

        lax.fori_loop(0, nch // 2, body, 0)
        plsc.subcore_barrier()
        pltpu.sync_copy(acc_sh.at[pl.ds(s * rpt, rpt)],
                        out_hbm.at[c, pl.ds(s * rpt, rpt)])

    zeros = jnp.zeros((rpt, f), jnp.float32)
    src3 = src.reshape(nw, nch, ch)
    dst3 = dst.reshape(nw, nch, ch)
    return scat_kernel(table, src3, dst3, zeros)


def kernel(mol_x, mol_edge_index, edge_index, W1m, b1m, W2m, b2m, W3m, b3m,
           W1, b1, W2, b2):
    n = mol_x.shape[0]
    src = edge_index[0]
    dst = edge_index[1]

    degw = _deg_call(dst, n)
    degt = jnp.transpose(degw[:, :, 0])
    kginv = _inv_call(degt)

    zs1 = _mol_call(kginv, mol_x, mol_edge_index,
                    W1m, b1m, W2m, b2m, W3m, b3m)
    t1p = _scatter_call(zs1, src, dst)
    zs2 = _mid_call(kginv, t1p, W1, b1, W2)
    t2p = _scatter_call(zs2, src, dst)
    return _fin_call(kginv, t2p, b2)


# speedup vs baseline: 61.6366x; 1.0021x over previous
import functools

import jax
import jax.numpy as jnp
from jax import lax
from jax.experimental import pallas as pl
from jax.experimental.pallas import tpu as pltpu
from jax.experimental.pallas import tpu_sc as plsc

_MB = 32
_SC_CORES = 2
_SC_TILES = 16
_DEG_W = 128


_SB = 8


def _mol_body(degt_ref, x_ref, ei_ref, w1_ref, b1_ref, w2_ref, b2_ref,
              w3_ref, b3_ref, zs_ref):
    mb, na, f = x_ref.shape
    me = ei_ref.shape[2]
    mbs = mb // _SB
    r = mbs * na
    e = mbs * me

    x = x_ref[...].reshape(mb * na, f)
    ei = ei_ref[...]

    eib = ei.astype(jnp.bfloat16)
    loc = (lax.broadcasted_iota(jnp.int32, (mbs, me, r), 2)
           - na * lax.broadcasted_iota(jnp.int32, (mbs, me, r), 0)
           ).astype(jnp.bfloat16)
    eye = (lax.broadcasted_iota(jnp.int32, (r, r), 0)
           == lax.broadcasted_iota(jnp.int32, (r, r), 1)).astype(jnp.float32)
    ps, invs = [], []
    for b in range(_SB):
        sb = eib[b * mbs:(b + 1) * mbs, 0, :]
        db = eib[b * mbs:(b + 1) * mbs, 1, :]
        oh_s = (sb[:, :, None] == loc).astype(jnp.int8).reshape(e, r)
        oh_d = (db[:, :, None] == loc).astype(jnp.int8).reshape(e, r)
        a = lax.dot_general(oh_d, oh_s, (((0,), (0,)), ((), ())),
                            preferred_element_type=jnp.int32
                            ).astype(jnp.float32)
        deg = jnp.sum(a, axis=1, keepdims=True) + 1.0
        inv = lax.rsqrt(deg)
        ps.append(((a + eye) * inv).astype(jnp.bfloat16))
        invs.append(inv)

    def agg(h):
        outs = []
        for b in range(_SB):
            hs = (invs[b] * h[b * r:(b + 1) * r, :]).astype(jnp.bfloat16)
            outs.append(lax.dot_general(ps[b], hs, (((1,), (0,)), ((), ())),
                                        preferred_element_type=jnp.float32))
        return jnp.concatenate(outs, axis=0)

    def mm(p, w_ref):
        return lax.dot_general(p.astype(jnp.bfloat16), w_ref[...],
                               (((1,), (0,)), ((), ())),
                               preferred_element_type=jnp.float32)

    h1 = jnp.maximum(mm(agg(x), w1_ref) + b1_ref[...], 0.0)
    h2 = jnp.maximum(mm(agg(h1), w2_ref) + b2_ref[...], 0.0)
    h3 = agg(mm(h2, w3_ref)) + b3_ref[...]

    kg = lax.rsqrt(degt_ref[:, 0:1] + degt_ref[:, 1:2] + 1.0)
    rows = jnp.concatenate(
        [jnp.min(h3[m * na:(m + 1) * na, :], axis=0, keepdims=True)
         * kg[m:m + 1, :] for m in range(mb)], axis=0)
    zs_ref[...] = rows


def _mol_call(degt, mol_x, mol_ei, w1, b1, w2, b2, w3, b3):
    n, na, f = mol_x.shape
    me = mol_ei.shape[2]
    c2 = w1.shape[1]
    return pl.pallas_call(
        _mol_body,
        grid=(n // _MB,),
        in_specs=[
            pl.BlockSpec((_MB, 2), lambda i: (i, 0)),
            pl.BlockSpec((_MB, na, f), lambda i: (i, 0, 0)),
            pl.BlockSpec((_MB, 2, me), lambda i: (i, 0, 0)),
            pl.BlockSpec((f, c2), lambda i: (0, 0)),
            pl.BlockSpec((1, c2), lambda i: (0, 0)),
            pl.BlockSpec((c2, c2), lambda i: (0, 0)),
            pl.BlockSpec((1, c2), lambda i: (0, 0)),
            pl.BlockSpec((c2, f), lambda i: (0, 0)),
            pl.BlockSpec((1, f), lambda i: (0, 0)),
        ],
        out_specs=pl.BlockSpec((_MB, f), lambda i: (i, 0)),
        out_shape=jax.ShapeDtypeStruct((n, f), jnp.float32),
    )(degt, mol_x, mol_ei,
      w1.astype(jnp.bfloat16), b1.reshape(1, -1),
      w2.astype(jnp.bfloat16), b2.reshape(1, -1),
      w3.astype(jnp.bfloat16), b3.reshape(1, -1))


def _inv_body(degt_ref, out_ref):
    d = degt_ref[:, 0:1] + degt_ref[:, 1:2] + 1.0
    out_ref[...] = lax.rsqrt(d)


def _inv_call(degt):
    n = degt.shape[0]
    return pl.pallas_call(
        _inv_body,
        out_shape=jax.ShapeDtypeStruct((n, 1), jnp.float32),
    )(degt)


def _mid_body(degt_ref, tp_ref, w1_ref, b1_ref, w2_ref, out_ref):
    inv = lax.rsqrt(degt_ref[:, 0:1] + degt_ref[:, 1:2] + 1.0)
    tp = tp_ref[...]
    t = tp[0] + tp[1]
    h1 = jnp.maximum(
        lax.dot_general(inv * t, w1_ref[...], (((1,), (0,)), ((), ())),
                        preferred_element_type=jnp.float32) + b1_ref[...], 0.0)
    out_ref[...] = inv * lax.dot_general(
        h1, w2_ref[...], (((1,), (0,)), ((), ())),
        preferred_element_type=jnp.float32)


def _mid_call(degt, tp, w1, b1, w2):
    _, n, f = tp.shape
    c2 = w1.shape[1]
    rb = 256
    return pl.pallas_call(
        _mid_body,
        grid=(n // rb,),
        in_specs=[
            pl.BlockSpec((rb, 2), lambda i: (i, 0)),
            pl.BlockSpec((2, rb, f), lambda i: (0, i, 0)),
            pl.BlockSpec((f, c2), lambda i: (0, 0)),
            pl.BlockSpec((1, c2), lambda i: (0, 0)),
            pl.BlockSpec((c2, f), lambda i: (0, 0)),
        ],
        out_specs=pl.BlockSpec((rb, f), lambda i: (i, 0)),
        out_shape=jax.ShapeDtypeStruct((n, f), jnp.float32),
    )(degt, tp, w1, b1.reshape(1, -1), w2)


def _fin_body(degt_ref, tp_ref, b2_ref, out_ref):
    inv = lax.rsqrt(degt_ref[:, 0:1] + degt_ref[:, 1:2] + 1.0)
    tp = tp_ref[...]
    out_ref[...] = inv * (tp[0] + tp[1]) + b2_ref[...]


def _fin_call(degt, tp, b2):
    _, n, f = tp.shape
    rb = 256
    return pl.pallas_call(
        _fin_body,
        grid=(n // rb,),
        in_specs=[
            pl.BlockSpec((rb, 2), lambda i: (i, 0)),
            pl.BlockSpec((2, rb, f), lambda i: (0, i, 0)),
            pl.BlockSpec((1, f), lambda i: (0, 0)),
        ],
        out_specs=pl.BlockSpec((rb, f), lambda i: (i, 0)),
        out_shape=jax.ShapeDtypeStruct((n, f), jnp.float32),
    )(degt, tp, b2.reshape(1, -1))


def _deg_call(dst, n):
    e_kg = dst.shape[0]
    nw = _SC_CORES * _SC_TILES
    ept = e_kg // nw
    ch = 128
    nch = ept // ch
    rpt = n // _SC_TILES
    mesh = plsc.VectorSubcoreMesh(core_axis_name="c", subcore_axis_name="s")

    @functools.partial(
        pl.kernel,
        out_type=jax.ShapeDtypeStruct((_SC_CORES, n, _DEG_W), jnp.float32),
        mesh=mesh,
        scratch_types=[
            pltpu.VMEM((nch, ch), jnp.int32),
            pltpu.VMEM((ch, _DEG_W), jnp.float32),
            pltpu.VMEM_SHARED((n, _DEG_W), jnp.float32),
            pltpu.SemaphoreType.DMA,
        ],
    )
    def deg_kernel(dst_hbm, zeros_hbm, ones_hbm, out_hbm, idx_v, ones_v,
                   acc_sh, sem):
        c = lax.axis_index("c")
        s = lax.axis_index("s")
        wid = c * _SC_TILES + s

        pltpu.sync_copy(ones_hbm, ones_v)
        pltpu.sync_copy(dst_hbm.at[wid], idx_v)
        pltpu.sync_copy(zeros_hbm, acc_sh.at[pl.ds(s * rpt, rpt)])
        plsc.subcore_barrier()

        def body(j, _):
            pltpu.async_copy(ones_v, acc_sh.at[idx_v.at[j]], sem, add=True)
            return 0

        lax.fori_loop(0, nch, body, 0)

        def drain(j, _):
            pltpu.make_async_copy(ones_hbm, ones_v, sem).wait()
            return 0

        lax.fori_loop(0, nch, drain, 0)
        plsc.subcore_barrier()
        pltpu.sync_copy(acc_sh.at[pl.ds(s * rpt, rpt)],
                        out_hbm.at[c, pl.ds(s * rpt, rpt)])

    zeros = jnp.zeros((rpt, _DEG_W), jnp.float32)
    ones = jnp.ones((ch, _DEG_W), jnp.float32)
    dst3 = dst.reshape(nw, nch, ch)
    return deg_kernel(dst3, zeros, ones)


def _scatter_call(table, src, dst):
    n, f = table.shape
    e_kg = src.shape[0]
    nw = _SC_CORES * _SC_TILES
    ept = e_kg // nw
    ch = 128
    nch = ept // ch
    rpt = n // _SC_TILES
    mesh = plsc.VectorSubcoreMesh(core_axis_name="c", subcore_axis_name="s")

    @functools.partial(
        pl.kernel,
        out_type=jax.ShapeDtypeStruct((_SC_CORES, n, f), jnp.float32),
        mesh=mesh,
        scratch_types=[
            pltpu.VMEM((nch, ch), jnp.int32),
            pltpu.VMEM((nch, ch), jnp.int32),
            pltpu.VMEM((ch, f), jnp.float32),
            pltpu.VMEM((ch, f), jnp.float32),
            pltpu.VMEM_SHARED((n, f), jnp.float32),
            pltpu.SemaphoreType.DMA,
            pltpu.SemaphoreType.DMA,
        ],
    )
    def scat_kernel(tab_hbm, src_hbm, dst_hbm, zeros_hbm, out_hbm,
                    si_v, di_v, rows0, rows1, acc_sh, sem0, sem1):
        c = lax.axis_index("c")
        s = lax.axis_index("s")
        wid = c * _SC_TILES + s

        pltpu.sync_copy(src_hbm.at[wid], si_v)
        pltpu.sync_copy(dst_hbm.at[wid], di_v)

        @pl.when(c == 0)
        def _():
            pltpu.sync_copy(tab_hbm.at[pl.ds(s * rpt, rpt)],
                            acc_sh.at[pl.ds(s * rpt, rpt)])

        @pl.when(c != 0)
        def _():
            pltpu.sync_copy(zeros_hbm, acc_sh.at[pl.ds(s * rpt, rpt)])

        plsc.subcore_barrier()

        def gather(j, buf, sem):
            pltpu.async_copy(tab_hbm.at[si_v.at[j]], buf, sem)

        def gwait(buf, sem):
            pltpu.make_async_copy(tab_hbm.at[si_v.at[0]], buf, sem).wait()

        gather(0, rows0, sem0)

        def body(jj, _):
            j0 = 2 * jj
            gather(j0 + 1, rows1, sem1)
            gwait(rows0, sem0)
            pltpu.sync_copy(rows0, acc_sh.at[di_v.at[j0]], add=True)

            @pl.when(j0 + 2 < nch)
            def _():
                gather(j0 + 2, rows0, sem0)

            gwait(rows1, sem1)
            pltpu.sync_copy(rows1, acc_sh.at[di_v.at[j0 + 1]], add=True)
            return 0

        lax.fori_loop(0, nch // 2, body, 0)
        plsc.subcore_barrier()
        pltpu.sync_copy(acc_sh.at[pl.ds(s * rpt, rpt)],
                        out_hbm.at[c, pl.ds(s * rpt, rpt)])

    zeros = jnp.zeros((rpt, f), jnp.float32)
    src3 = src.reshape(nw, nch, ch)
    dst3 = dst.reshape(nw, nch, ch)
    return scat_kernel(table, src3, dst3, zeros)


def kernel(mol_x, mol_edge_index, edge_index, W1m, b1m, W2m, b2m, W3m, b3m,
           W1, b1, W2, b2):
    n = mol_x.shape[0]
    src = edge_index[0]
    dst = edge_index[1]

    degw = _deg_call(dst, n)
    degt = jnp.transpose(degw[:, :, 0])

    zs1 = _mol_call(degt, mol_x, mol_edge_index,
                    W1m, b1m, W2m, b2m, W3m, b3m)
    t1p = _scatter_call(zs1, src, dst)
    zs2 = _mid_call(degt, t1p, W1, b1, W2)
    t2p = _scatter_call(zs2, src, dst)
    return _fin_call(degt, t2p, b2)


# speedup vs baseline: 61.6594x; 1.0004x over previous
import functools

import jax
import jax.numpy as jnp
from jax import lax
from jax.experimental import pallas as pl
from jax.experimental.pallas import tpu as pltpu
from jax.experimental.pallas import tpu_sc as plsc

_MB = 32
_SC_CORES = 2
_SC_TILES = 16
_DEG_W = 128


_SB = 8


def _mol_body(degt_ref, x_ref, ei_ref, w1_ref, b1_ref, w2_ref, b2_ref,
              w3_ref, b3_ref, zs_ref):
    mb, na, f = x_ref.shape
    me = ei_ref.shape[2]
    mbs = mb // _SB
    r = mbs * na
    e = mbs * me

    x = x_ref[...].reshape(mb * na, f)
    ei = ei_ref[...]

    eib = ei.astype(jnp.bfloat16)
    loc = (lax.broadcasted_iota(jnp.int32, (mbs, me, r), 2)
           - na * lax.broadcasted_iota(jnp.int32, (mbs, me, r), 0)
           ).astype(jnp.bfloat16)
    eye = (lax.broadcasted_iota(jnp.int32, (r, r), 0)
           == lax.broadcasted_iota(jnp.int32, (r, r), 1)).astype(jnp.float32)
    ps, invs = [], []
    for b in range(_SB):
        sb = eib[b * mbs:(b + 1) * mbs, 0, :]
        db = eib[b * mbs:(b + 1) * mbs, 1, :]
        oh_s = (sb[:, :, None] == loc).astype(jnp.int8).reshape(e, r)
        oh_d = (db[:, :, None] == loc).astype(jnp.int8).reshape(e, r)
        a = lax.dot_general(oh_d, oh_s, (((0,), (0,)), ((), ())),
                            preferred_element_type=jnp.int32
                            ).astype(jnp.float32)
        deg = jnp.sum(a, axis=1, keepdims=True) + 1.0
        inv = lax.rsqrt(deg)
        ps.append(((a + eye) * inv).astype(jnp.bfloat16))
        invs.append(inv)

    def agg(h):
        outs = []
        for b in range(_SB):
            hs = (invs[b] * h[b * r:(b + 1) * r, :]).astype(jnp.bfloat16)
            outs.append(lax.dot_general(ps[b], hs, (((1,), (0,)), ((), ())),
                                        preferred_element_type=jnp.float32))
        return jnp.concatenate(outs, axis=0)

    def mm(p, w_ref):
        return lax.dot_general(p.astype(jnp.bfloat16), w_ref[...],
                               (((1,), (0,)), ((), ())),
                               preferred_element_type=jnp.float32)

    h1 = jnp.maximum(mm(agg(x), w1_ref) + b1_ref[...], 0.0)
    h2 = jnp.maximum(mm(agg(h1), w2_ref) + b2_ref[...], 0.0)
    h3 = agg(mm(h2, w3_ref)) + b3_ref[...]

    kg = lax.rsqrt(degt_ref[:, 0:1] + degt_ref[:, 1:2] + 1.0)
    rows = jnp.concatenate(
        [jnp.min(h3[m * na:(m + 1) * na, :], axis=0, keepdims=True)
         * kg[m:m + 1, :] for m in range(mb)], axis=0)
    zs_ref[...] = rows


def _mol_call(degt, mol_x, mol_ei, w1, b1, w2, b2, w3, b3):
    n, na, f = mol_x.shape
    me = mol_ei.shape[2]
    c2 = w1.shape[1]
    return pl.pallas_call(
        _mol_body,
        grid=(n // _MB,),
        in_specs=[
            pl.BlockSpec((_MB, 2), lambda i: (i, 0)),
            pl.BlockSpec((_MB, na, f), lambda i: (i, 0, 0)),
            pl.BlockSpec((_MB, 2, me), lambda i: (i, 0, 0)),
            pl.BlockSpec((f, c2), lambda i: (0, 0)),
            pl.BlockSpec((1, c2), lambda i: (0, 0)),
            pl.BlockSpec((c2, c2), lambda i: (0, 0)),
            pl.BlockSpec((1, c2), lambda i: (0, 0)),
            pl.BlockSpec((c2, f), lambda i: (0, 0)),
            pl.BlockSpec((1, f), lambda i: (0, 0)),
        ],
        out_specs=pl.BlockSpec((_MB, f), lambda i: (i, 0)),
        out_shape=jax.ShapeDtypeStruct((n, f), jnp.float32),
    )(degt, mol_x, mol_ei,
      w1.astype(jnp.bfloat16), b1.reshape(1, -1),
      w2.astype(jnp.bfloat16), b2.reshape(1, -1),
      w3.astype(jnp.bfloat16), b3.reshape(1, -1))


def _mid_body(degt_ref, tp_ref, w1_ref, b1_ref, w2_ref, out_ref):
    inv = lax.rsqrt(degt_ref[:, 0:1] + degt_ref[:, 1:2] + 1.0)
    tp = tp_ref[...]
    t = tp[0] + tp[1]
    h1 = jnp.maximum(
        lax.dot_general(inv * t, w1_ref[...], (((1,), (0,)), ((), ())),
                        preferred_element_type=jnp.float32) + b1_ref[...], 0.0)
    out_ref[...] = inv * lax.dot_general(
        h1, w2_ref[...], (((1,), (0,)), ((), ())),
        preferred_element_type=jnp.float32)


def _mid_call(degt, tp, w1, b1, w2):
    _, n, f = tp.shape
    c2 = w1.shape[1]
    rb = 256
    return pl.pallas_call(
        _mid_body,
        grid=(n // rb,),
        in_specs=[
            pl.BlockSpec((rb, 2), lambda i: (i, 0)),
            pl.BlockSpec((2, rb, f), lambda i: (0, i, 0)),
            pl.BlockSpec((f, c2), lambda i: (0, 0)),
            pl.BlockSpec((1, c2), lambda i: (0, 0)),
            pl.BlockSpec((c2, f), lambda i: (0, 0)),
        ],
        out_specs=pl.BlockSpec((rb, f), lambda i: (i, 0)),
        out_shape=jax.ShapeDtypeStruct((n, f), jnp.float32),
    )(degt, tp, w1, b1.reshape(1, -1), w2)


def _fin_body(degt_ref, tp_ref, b2_ref, out_ref):
    inv = lax.rsqrt(degt_ref[:, 0:1] + degt_ref[:, 1:2] + 1.0)
    tp = tp_ref[...]
    out_ref[...] = inv * (tp[0] + tp[1]) + b2_ref[...]


def _fin_call(degt, tp, b2):
    _, n, f = tp.shape
    rb = 256
    return pl.pallas_call(
        _fin_body,
        grid=(n // rb,),
        in_specs=[
            pl.BlockSpec((rb, 2), lambda i: (i, 0)),
            pl.BlockSpec((2, rb, f), lambda i: (0, i, 0)),
            pl.BlockSpec((1, f), lambda i: (0, 0)),
        ],
        out_specs=pl.BlockSpec((rb, f), lambda i: (i, 0)),
        out_shape=jax.ShapeDtypeStruct((n, f), jnp.float32),
    )(degt, tp, b2.reshape(1, -1))


def _deg_call(dst, n):
    e_kg = dst.shape[0]
    nw = _SC_CORES * _SC_TILES
    ept = e_kg // nw
    ch = 128
    nch = ept // ch
    rpt = n // _SC_TILES
    mesh = plsc.VectorSubcoreMesh(core_axis_name="c", subcore_axis_name="s")

    @functools.partial(
        pl.kernel,
        out_type=jax.ShapeDtypeStruct((_SC_CORES, n, _DEG_W), jnp.float32),
        mesh=mesh,
        scratch_types=[
            pltpu.VMEM((nch, ch), jnp.int32),
            pltpu.VMEM((ch, _DEG_W), jnp.float32),
            pltpu.VMEM_SHARED((n, _DEG_W), jnp.float32),
            pltpu.SemaphoreType.DMA,
        ],
    )
    def deg_kernel(dst_hbm, zeros_hbm, ones_hbm, out_hbm, idx_v, ones_v,
                   acc_sh, sem):
        c = lax.axis_index("c")
        s = lax.axis_index("s")
        wid = c * _SC_TILES + s

        pltpu.sync_copy(ones_hbm, ones_v)
        pltpu.sync_copy(dst_hbm.at[wid], idx_v)
        pltpu.sync_copy(zeros_hbm, acc_sh.at[pl.ds(s * rpt, rpt)])
        plsc.subcore_barrier()

        def body(j, _):
            pltpu.async_copy(ones_v, acc_sh.at[idx_v.at[j]], sem, add=True)
            return 0

        lax.fori_loop(0, nch, body, 0)

        def drain(j, _):
            pltpu.make_async_copy(ones_hbm, ones_v, sem).wait()
            return 0

        lax.fori_loop(0, nch, drain, 0)
        plsc.subcore_barrier()
        pltpu.sync_copy(acc_sh.at[pl.ds(s * rpt, rpt)],
                        out_hbm.at[c, pl.ds(s * rpt, rpt)])

    zeros = jnp.zeros((rpt, _DEG_W), jnp.float32)
    ones = jnp.ones((ch, _DEG_W), jnp.float32)
    dst3 = dst.reshape(nw, nch, ch)
    return deg_kernel(dst3, zeros, ones)


def _scatter_call(table, src, dst):
    n, f = table.shape
    e_kg = src.shape[0]
    nw = _SC_CORES * _SC_TILES
    ept = e_kg // nw
    ch = 128
    nch = ept // ch
    rpt = n // _SC_TILES
    mesh = plsc.VectorSubcoreMesh(core_axis_name="c", subcore_axis_name="s")

    @functools.partial(
        pl.kernel,
        out_type=jax.ShapeDtypeStruct((_SC_CORES, n, f), jnp.float32),
        mesh=mesh,
        scratch_types=[
            pltpu.VMEM((nch, ch), jnp.int32),
            pltpu.VMEM((nch, ch), jnp.int32),
            pltpu.VMEM((ch, f), jnp.float32),
            pltpu.VMEM((ch, f), jnp.float32),
            pltpu.VMEM_SHARED((n, f), jnp.float32),
            pltpu.SemaphoreType.DMA,
            pltpu.SemaphoreType.DMA,
        ],
    )
    def scat_kernel(tab_hbm, src_hbm, dst_hbm, zeros_hbm, out_hbm,
                    si_v, di_v, rows0, rows1, acc_sh, sem0, sem1):
        c = lax.axis_index("c")
        s = lax.axis_index("s")
        wid = c * _SC_TILES + s

        pltpu.sync_copy(src_hbm.at[wid], si_v)
        pltpu.sync_copy(dst_hbm.at[wid], di_v)

        @pl.when(c == 0)
        def _():
            pltpu.sync_copy(tab_hbm.at[pl.ds(s * rpt, rpt)],
                            acc_sh.at[pl.ds(s * rpt, rpt)])

        @pl.when(c != 0)
        def _():
            pltpu.sync_copy(zeros_hbm, acc_sh.at[pl.ds(s * rpt, rpt)])

        plsc.subcore_barrier()

        def gather(j, buf, sem):
            pltpu.async_copy(tab_hbm.at[si_v.at[j]], buf, sem)

        def gwait(buf, sem):
            pltpu.make_async_copy(tab_hbm.at[si_v.at[0]], buf, sem).wait()

        gather(0, rows0, sem0)

        def body(jj, _):
            j0 = 2 * jj
            gather(j0 + 1, rows1, sem1)
            gwait(rows0, sem0)
            pltpu.sync_copy(rows0, acc_sh.at[di_v.at[j0]], add=True)

            @pl.when(j0 + 2 < nch)
            def _():
                gather(j0 + 2, rows0, sem0)

            gwait(rows1, sem1)
            pltpu.sync_copy(rows1, acc_sh.at[di_v.at[j0 + 1]], add=True)
            return 0

        lax.fori_loop(0, nch // 2, body, 0)
        plsc.subcore_barrier()
        pltpu.sync_copy(acc_sh.at[pl.ds(s * rpt, rpt)],
                        out_hbm.at[c, pl.ds(s * rpt, rpt)])

    zeros = jnp.zeros((rpt, f), jnp.float32)
    src3 = src.reshape(nw, nch, ch)
    dst3 = dst.reshape(nw, nch, ch)
    return scat_kernel(table, src3, dst3, zeros)


def kernel(mol_x, mol_edge_index, edge_index, W1m, b1m, W2m, b2m, W3m, b3m,
           W1, b1, W2, b2):
    n = mol_x.shape[0]
    src = edge_index[0]
    dst = edge_index[1]

    degw = _deg_call(dst, n)
    degt = jnp.transpose(degw[:, :, 0])

    zs1 = _mol_call(degt, mol_x, mol_edge_index,
                    W1m, b1m, W2m, b2m, W3m, b3m)
    t1p = _scatter_call(zs1, src, dst)
    zs2 = _mid_call(degt, t1p, W1, b1, W2)
    t2p = _scatter_call(zs2, src, dst)
    return _fin_call(degt, t2p, b2)
